# Initial kernel scaffold; baseline (speedup 1.0000x reference)
import jax
import jax.numpy as jnp
from jax.experimental import pallas as pl


def kernel(positions, d_table, h_table, w_table, proj_w, proj_b):
    raise NotImplementedError("write your pallas kernel here")


# speedup vs baseline: 2.0263x; 2.0263x over previous
import functools

import jax
import jax.numpy as jnp
from jax import lax
from jax.experimental import pallas as pl
from jax.experimental.pallas import tpu as pltpu
from jax.experimental.pallas import tpu_sc as plsc

_EMBED = 384
_NPOS = 64
_D3 = 128
_LANES = 16

_NC, _NS = 2, 16
_NW = _NC * _NS


def _fold_body(d_ref, h_ref, w_ref, wt_ref, b_ref, out_ref):
    dot = functools.partial(
        jnp.dot,
        preferred_element_type=jnp.float32,
        precision=lax.Precision.HIGHEST,
    )
    bias = b_ref[0, :]
    out_ref[0:_NPOS, :] = dot(d_ref[...], wt_ref[0:_D3, :]) + bias[None, :]
    out_ref[_NPOS : 2 * _NPOS, :] = dot(h_ref[...], wt_ref[_D3 : 2 * _D3, :])
    out_ref[2 * _NPOS : 3 * _NPOS, :] = dot(w_ref[...], wt_ref[2 * _D3 :, :])


def _build_fused_table(d_table, h_table, w_table, proj_w, proj_b):
    return pl.pallas_call(
        _fold_body,
        out_shape=jax.ShapeDtypeStruct((3 * _NPOS, _EMBED), jnp.float32),
    )(d_table, h_table, w_table, proj_w.T, proj_b.reshape(1, _EMBED))


def _make_sc_kernel(n_tok, chunk):
    per_w = n_tok // _NW
    n_chunks = per_w // chunk
    mesh = plsc.VectorSubcoreMesh(core_axis_name="c", subcore_axis_name="s")

    @functools.partial(
        pl.kernel,
        out_type=jax.ShapeDtypeStruct((n_tok, _EMBED), jnp.float32),
        mesh=mesh,
        scratch_types=[
            pltpu.VMEM((3, chunk), jnp.int32),
            pltpu.VMEM((3, chunk, _EMBED), jnp.float32),
            pltpu.SemaphoreType.DMA,
        ],
    )
    def sc_kernel(p_hbm, idx_hbm, out_hbm, idx_v, bufs, sem):
        wid = lax.axis_index("s") * _NC + lax.axis_index("c")
        base = wid * per_w

        def chunk_body(g, carry):
            row0 = base + g * chunk
            for t in range(3):
                pltpu.sync_copy(idx_hbm.at[t, pl.ds(row0, chunk)], idx_v.at[t])
            cps = [
                pltpu.async_copy(p_hbm.at[idx_v.at[t]], bufs.at[t], sem)
                for t in range(3)
            ]
            for cp in cps:
                cp.wait()

            def add_body(t, carry2):
                for c in range(_EMBED // _LANES):
                    sl = pl.ds(c * _LANES, _LANES)
                    bufs[0, t, sl] = bufs[0, t, sl] + bufs[1, t, sl] + bufs[2, t, sl]
                return carry2

            lax.fori_loop(0, chunk, add_body, 0)
            pltpu.sync_copy(bufs.at[0], out_hbm.at[pl.ds(row0, chunk)])
            return carry

        lax.fori_loop(0, n_chunks, chunk_body, 0)

    return sc_kernel


def kernel(positions, d_table, h_table, w_table, proj_w, proj_b):
    b, n, _ = positions.shape
    n_tok = b * n
    pos = jnp.clip(positions.astype(jnp.int32), 0, _NPOS - 1).reshape(n_tok, 3)
    idx = (pos + jnp.array([0, _NPOS, 2 * _NPOS], dtype=jnp.int32)).T

    fused = _build_fused_table(d_table, h_table, w_table, proj_w, proj_b)
    out = _make_sc_kernel(n_tok, 64)(fused, idx)
    return out.reshape(b, n, _EMBED)
